# baseline (speedup 1.0000x reference)
import functools

import jax
import jax.numpy as jnp
from jax import lax
from jax.experimental import pallas as pl
from jax.experimental.pallas import tpu as pltpu
from jax.experimental.pallas import tpu_sc as plsc

_NC = 2
_NS = 16
_NW = _NC * _NS
_GRAIN = 1024


def _mesh():
    return plsc.VectorSubcoreMesh(core_axis_name="c", subcore_axis_name="s")


def _pack2(a, b):
    hi = jnp.uint32(0xFFFF0000)
    aw = lax.bitcast_convert_type(a, jnp.uint32)
    bw = lax.bitcast_convert_type(b, jnp.uint32)
    return lax.bitcast_convert_type((bw & hi) | (aw >> 16), jnp.float32)


def _unpack_lo(w):
    ww = lax.bitcast_convert_type(w, jnp.uint32)
    return lax.bitcast_convert_type(ww << 16, jnp.float32)


def _unpack_hi(w):
    ww = lax.bitcast_convert_type(w, jnp.uint32)
    return lax.bitcast_convert_type(ww & jnp.uint32(0xFFFF0000), jnp.float32)


def _proj_l1_body(x_ref, wl_ref, bl_ref, wr_ref, br_ref, a2_ref,
                  xl_ref, xr_ref, xrf_ref):
    x = x_ref[...]
    xl = jnp.dot(x, wl_ref[...], preferred_element_type=jnp.float32) + bl_ref[...]
    xr = jnp.dot(x, wr_ref[...], preferred_element_type=jnp.float32) + br_ref[...]
    t = xl + xr
    lk = jnp.where(t >= 0, t, 0.2 * t)
    s128 = jnp.dot(lk, a2_ref[...], preferred_element_type=jnp.float32)
    xl_ref[...] = _pack2(xl[:, :256], xl[:, 256:])
    xr_ref[:, :256] = _pack2(xr[:, :256], xr[:, 256:])
    xr_ref[:, 256:384] = _pack2(s128, jnp.zeros_like(s128))
    xrf_ref[...] = xr


def _proj_l2_body(x_ref, wl_ref, bl_ref, wr_ref, br_ref, a2_ref, rp_ref,
                  xl_ref, xr_ref, xrf_ref):
    x = x_ref[...]
    xl = jnp.dot(x, wl_ref[...], preferred_element_type=jnp.float32) + bl_ref[...]
    xr = jnp.dot(x, wr_ref[...], preferred_element_type=jnp.float32) + br_ref[...]
    t = xl + xr
    lk = jnp.where(t >= 0, t, 0.2 * t)
    rp = rp_ref[...]
    xl_ref[...] = jnp.dot(xl, rp, preferred_element_type=jnp.float32)
    xr_ref[...] = (jnp.dot(xr, rp, preferred_element_type=jnp.float32)
                   + jnp.dot(lk, a2_ref[...],
                             preferred_element_type=jnp.float32))
    xrf_ref[...] = xr


def _dense_proj(x, Wl, bl, Wr, br, a2, rpad, xw, w, block=2000):
    n, d = x.shape
    hc = Wl.shape[1]
    grid = n // block
    body = _proj_l1_body if rpad is None else _proj_l2_body
    in_specs = [
        pl.BlockSpec((block, d), lambda i: (i, 0)),
        pl.BlockSpec((d, hc), lambda i: (0, 0)),
        pl.BlockSpec((1, hc), lambda i: (0, 0)),
        pl.BlockSpec((d, hc), lambda i: (0, 0)),
        pl.BlockSpec((1, hc), lambda i: (0, 0)),
        pl.BlockSpec((hc, 128), lambda i: (0, 0)),
    ]
    args = [x, Wl, bl.reshape(1, hc), Wr, br.reshape(1, hc), a2]
    if rpad is not None:
        in_specs.append(pl.BlockSpec((hc, xw), lambda i: (0, 0)))
        args.append(rpad)
    return pl.pallas_call(
        body,
        grid=(grid,),
        in_specs=in_specs,
        out_specs=[
            pl.BlockSpec((block, xw), lambda i: (i, 0)),
            pl.BlockSpec((block, w), lambda i: (i, 0)),
            pl.BlockSpec((block, hc), lambda i: (i, 0)),
        ],
        out_shape=[
            jax.ShapeDtypeStruct((n, xw), jnp.float32),
            jax.ShapeDtypeStruct((n, w), jnp.float32),
            jax.ShapeDtypeStruct((n, hc), jnp.float32),
        ],
    )(*args)


def _edge_l1_body(e2, b2, xl_ref, xr_ref, a2a_ref, a2b_ref, pa_ref, pb_ref,
                  p2_ref, out_ref):
    xlw = xl_ref[...]
    xrw = xr_ref[:, :256]
    ua = _unpack_lo(xlw) + _unpack_lo(xrw)
    ub = _unpack_hi(xlw) + _unpack_hi(xrw)
    s = _unpack_lo(xr_ref[:, 256:384])[:, :8]
    lka = jnp.where(ua >= 0, ua, 0.2 * ua)
    lkb = jnp.where(ub >= 0, ub, 0.2 * ub)
    alpha = (jnp.dot(lka, a2a_ref[...], preferred_element_type=jnp.float32)
             + jnp.dot(lkb, a2b_ref[...],
                       preferred_element_type=jnp.float32))[:, :8]
    row = pl.program_id(0) * b2 + lax.broadcasted_iota(jnp.int32, (b2, 1), 0)
    aex = jnp.where(row < e2, jnp.exp(alpha - s), 0.0)
    out_ref[:, :256] = ua * jnp.dot(aex, pa_ref[...],
                                    preferred_element_type=jnp.float32)
    out_ref[:, 256:512] = ub * jnp.dot(aex, pb_ref[...],
                                       preferred_element_type=jnp.float32)
    out_ref[:, 512:640] = jnp.dot(aex, p2_ref[...],
                                  preferred_element_type=jnp.float32)


def _edge_l2_body(e2, b2, xl_ref, xr_ref, a2_ref, p_ref, p2_ref, out_ref):
    u = xl_ref[...] + xr_ref[...]
    lk0 = u[:, :64]
    lk = jnp.where(lk0 >= 0, lk0, 0.2 * lk0)
    alpha = jnp.dot(lk, a2_ref[...], preferred_element_type=jnp.float32)[:, :1]
    s = xr_ref[:, 64:65]
    row = pl.program_id(0) * b2 + lax.broadcasted_iota(jnp.int32, (b2, 1), 0)
    aex = jnp.where(row < e2, jnp.exp(alpha - s), 0.0)
    out_ref[...] = (u * jnp.dot(aex, p_ref[...],
                                preferred_element_type=jnp.float32)
                    + jnp.dot(aex, p2_ref[...],
                              preferred_element_type=jnp.float32))


def _edge_math(xl_src, xr_g, mats, e2, heads, outw, b2=2048):
    e2p, xw = xl_src.shape
    w = xr_g.shape[1]
    grid = e2p // b2
    body = functools.partial(
        _edge_l1_body if heads == 8 else _edge_l2_body, e2, b2)
    return pl.pallas_call(
        body,
        grid=(grid,),
        in_specs=[
            pl.BlockSpec((b2, xw), lambda i: (i, 0)),
            pl.BlockSpec((b2, w), lambda i: (i, 0)),
        ] + [pl.BlockSpec(m.shape, lambda i: (0, 0)) for m in mats],
        out_specs=pl.BlockSpec((b2, outw), lambda i: (i, 0)),
        out_shape=jax.ShapeDtypeStruct((e2p, outw), jnp.float32),
    )(xl_src, xr_g, *mats)


def _sc_gather(table, idx2, pw):
    _, F = table.shape
    e2p = idx2.shape[0] * 128
    B = 128
    nb = pw // B
    irw = pw // 128

    @functools.partial(
        pl.kernel, mesh=_mesh(),
        out_type=jax.ShapeDtypeStruct((e2p, F), table.dtype),
        scratch_types=(
            [pltpu.VMEM((irw, 128), jnp.int32)]
            + [pltpu.VMEM((B, F), table.dtype) for _ in range(2)]
            + [pltpu.SemaphoreType.DMA for _ in range(4)]
        ),
    )
    def kfn(table_hbm, idx_hbm, out_hbm, idx_all, r0, r1, g0, g1, w0, w1):
        rows = [r0, r1]
        gsem = [g0, g1]
        wsem = [w0, w1]
        w = lax.axis_index("s") * _NC + lax.axis_index("c")
        base = w * pw
        pltpu.sync_copy(idx_hbm.at[pl.ds(w * irw, irw)], idx_all)

        def gstart(i, sl):
            pltpu.async_copy(table_hbm.at[idx_all.at[i]], rows[sl], gsem[sl])

        def gwait(sl):
            pltpu.make_async_copy(table_hbm.at[pl.ds(0, B)],
                                  rows[sl], gsem[sl]).wait()

        def wstart(i, sl):
            pltpu.async_copy(rows[sl], out_hbm.at[pl.ds(base + i * B, B)],
                             wsem[sl])

        def wwait(sl):
            pltpu.make_async_copy(table_hbm.at[pl.ds(0, B)],
                                  rows[sl], wsem[sl]).wait()

        gstart(0, 0)
        gstart(1, 1)

        @pl.loop(0, nb // 2 - 1)
        def _(t):
            i0 = 2 * t
            gwait(0)
            wstart(i0, 0)
            gwait(1)
            wstart(i0 + 1, 1)
            wwait(0)
            gstart(i0 + 2, 0)
            wwait(1)
            gstart(i0 + 3, 1)

        gwait(0)
        wstart(nb - 2, 0)
        gwait(1)
        wstart(nb - 1, 1)
        wwait(0)
        wwait(1)

    return kfn(table, idx2)


def _acc_copy(src, dst, s, nn):
    slab = (nn // _NS) & ~7
    pltpu.sync_copy(src.at[pl.ds(s * slab, slab)], dst.at[pl.ds(s * slab, slab)])
    tail = nn - _NS * slab
    if tail:
        @pl.when(s == 0)
        def _():
            pltpu.sync_copy(src.at[pl.ds(_NS * slab, tail)],
                            dst.at[pl.ds(_NS * slab, tail)])


def _sc_scatter(v, idx2, zc, nch, pw):
    nn = zc.shape[0]
    B = 128
    nb = pw // B
    irw = pw // 128

    @functools.partial(
        pl.kernel, mesh=_mesh(),
        out_type=jax.ShapeDtypeStruct((_NC * nch, nn, 128), jnp.float32),
        scratch_types=(
            [pltpu.VMEM((irw, 128), jnp.int32)]
            + [pltpu.VMEM((B, 128), jnp.float32) for _ in range(2)]
            + [pltpu.VMEM_SHARED((nn, 128), jnp.float32)]
            + [pltpu.SemaphoreType.DMA for _ in range(4)]
        ),
    )
    def kfn(v_hbm, idx_hbm, zc_hbm, agg_hbm, idx_all,
            r0, r1, acc_sh, l0, l1, s0, s1):
        rows = [r0, r1]
        lsem = [l0, l1]
        ssem = [s0, s1]
        c = lax.axis_index("c")
        s = lax.axis_index("s")
        w = s * _NC + c
        base = w * pw
        pltpu.sync_copy(idx_hbm.at[pl.ds(w * irw, irw)], idx_all)

        for ch in range(nch):
            _acc_copy(zc_hbm, acc_sh, s, nn)
            plsc.subcore_barrier()

            def lstart(i, sl):
                pltpu.async_copy(
                    v_hbm.at[pl.ds(base + i * B, B), pl.ds(ch * 128, 128)],
                    rows[sl], lsem[sl])

            def lwait(sl):
                pltpu.make_async_copy(
                    v_hbm.at[pl.ds(base, B), pl.ds(ch * 128, 128)],
                    rows[sl], lsem[sl]).wait()

            def sstart(i, sl):
                pltpu.async_copy(rows[sl], acc_sh.at[idx_all.at[i]],
                                 ssem[sl], add=True)

            def swait(sl):
                pltpu.make_async_copy(
                    v_hbm.at[pl.ds(base, B), pl.ds(ch * 128, 128)],
                    rows[sl], ssem[sl]).wait()

            lstart(0, 0)
            lstart(1, 1)

            @pl.loop(0, nb // 2 - 1)
            def _(t):
                i0 = 2 * t
                lwait(0)
                sstart(i0, 0)
                lwait(1)
                sstart(i0 + 1, 1)
                swait(0)
                lstart(i0 + 2, 0)
                swait(1)
                lstart(i0 + 3, 1)

            lwait(0)
            sstart(nb - 2, 0)
            lwait(1)
            sstart(nb - 1, 1)
            swait(0)
            swait(1)

            plsc.subcore_barrier()
            _acc_copy(acc_sh, agg_hbm.at[c * nch + ch], s, nn)

    return kfn(v, idx2, zc)


def _gat_layer(x, src2, dst2, e2, Wl, bl, Wr, br, att, bias, heads, ch):
    n = x.shape[0]
    hc = heads * ch
    e2p = src2.shape[0] * 128
    pw = e2p // _NW
    src = src2
    dst = dst2
    augw = -(hc + heads) % 128 + hc + heads

    af = att.reshape(hc)
    hrep = jnp.repeat(jnp.arange(heads), ch)
    if heads == 8:
        a2_proj = jnp.zeros((hc, 128), jnp.float32).at[
            jnp.arange(hc), hrep].set(af)
        mats = [a2_proj[:256], a2_proj[256:],
                jnp.repeat(jnp.eye(heads, dtype=jnp.float32), ch, axis=1)[:, :256],
                jnp.repeat(jnp.eye(heads, dtype=jnp.float32), ch, axis=1)[:, 256:],
                jnp.eye(heads, 128, dtype=jnp.float32)]
        rpad = None
        xw, ww = 256, 384
    else:
        a2_proj = jnp.zeros((hc, 128), jnp.float32).at[
            jnp.arange(hc), 64].set(af)
        a2_edge = jnp.zeros((hc, 128), jnp.float32).at[
            jnp.arange(hc), 0].set(af)
        mats = [a2_edge,
                (jnp.arange(128) < 64).astype(jnp.float32).reshape(1, 128),
                jnp.zeros((1, 128), jnp.float32).at[0, 64].set(1.0)]
        rpad = jnp.eye(hc, 128, dtype=jnp.float32)
        xw, ww = 128, 128

    xl_t, xr_aug, xr = _dense_proj(x, Wl, bl, Wr, br, a2_proj, rpad, xw, ww)

    xl_src = _sc_gather(xl_t, src, pw)
    xr_g = _sc_gather(xr_aug, dst, pw)

    v_aug = _edge_math(xl_src, xr_g, mats, e2, heads, augw)

    nch = augw // 128
    zc = jnp.zeros((n, 128), jnp.float32)
    parts = _sc_scatter(v_aug, dst2, zc, nch, pw)

    agg = (parts[:nch] + parts[nch:]).transpose(1, 0, 2).reshape(n, nch * 128)
    aggu = agg[:, :hc]
    den = agg[:, hc:hc + heads]
    out = (aggu.reshape(n, heads, ch) / den[..., None]
           - xr.reshape(n, heads, ch)).reshape(n, hc)
    return out + bias


def _bn_gelu(x, g, b):
    mu = x.mean(0)
    var = x.var(0)
    return jax.nn.gelu(g * (x - mu) * jax.lax.rsqrt(var + 1e-5) + b,
                       approximate=False)


def kernel(x, edge_index, Wl1, bl1, Wr1, br1, att1, bias1, g1, be1,
           Wl2, bl2, Wr2, br2, att2, bias2, g2, be2, Wc, bc):
    n = x.shape[0]
    e = edge_index.shape[1]
    e2 = e + n
    grain = _NW * _GRAIN
    e2p = ((e2 + grain - 1) // grain) * grain

    loop = jnp.arange(n, dtype=edge_index.dtype)
    pad = e2p - e2
    src2 = jnp.pad(jnp.concatenate([edge_index[0], loop]),
                   (0, pad)).reshape(e2p // 128, 128)
    dst2 = jnp.pad(jnp.concatenate([edge_index[1], loop]),
                   (0, pad)).reshape(e2p // 128, 128)

    h = _gat_layer(x, src2, dst2, e2, Wl1, bl1, Wr1, br1, att1, bias1, 8, 64)
    h = _bn_gelu(h, g1, be1)
    h = _gat_layer(h, src2, dst2, e2, Wl2, bl2, Wr2, br2, att2, bias2, 1, 64)
    h = _bn_gelu(h, g2, be2)
    return jax.nn.log_softmax(h @ Wc + bc, axis=1)


# speedup vs baseline: 1.0349x; 1.0349x over previous
import functools

import jax
import jax.numpy as jnp
from jax import lax
from jax.experimental import pallas as pl
from jax.experimental.pallas import tpu as pltpu
from jax.experimental.pallas import tpu_sc as plsc

_NC = 2
_NS = 16
_NW = _NC * _NS
_GRAIN = 1024


def _mesh():
    return plsc.VectorSubcoreMesh(core_axis_name="c", subcore_axis_name="s")


def _pack2(a, b):
    hi = jnp.uint32(0xFFFF0000)
    aw = lax.bitcast_convert_type(a, jnp.uint32)
    bw = lax.bitcast_convert_type(b, jnp.uint32)
    return lax.bitcast_convert_type((bw & hi) | (aw >> 16), jnp.float32)


def _unpack_lo(w):
    ww = lax.bitcast_convert_type(w, jnp.uint32)
    return lax.bitcast_convert_type(ww << 16, jnp.float32)


def _unpack_hi(w):
    ww = lax.bitcast_convert_type(w, jnp.uint32)
    return lax.bitcast_convert_type(ww & jnp.uint32(0xFFFF0000), jnp.float32)


def _proj_l1_body(x_ref, wl_ref, bl_ref, wr_ref, br_ref, a2_ref,
                  xl_ref, xr_ref, xrf_ref):
    x = x_ref[...]
    xl = jnp.dot(x, wl_ref[...], preferred_element_type=jnp.float32) + bl_ref[...]
    xr = jnp.dot(x, wr_ref[...], preferred_element_type=jnp.float32) + br_ref[...]
    t = xl + xr
    lk = jnp.where(t >= 0, t, 0.2 * t)
    s128 = jnp.dot(lk, a2_ref[...], preferred_element_type=jnp.float32)
    xl_ref[...] = _pack2(xl[:, :256], xl[:, 256:])
    xr_ref[:, :256] = _pack2(xr[:, :256], xr[:, 256:])
    xr_ref[:, 256:384] = _pack2(s128, jnp.zeros_like(s128))
    xrf_ref[...] = xr


def _proj_l2_body(x_ref, wl_ref, bl_ref, wr_ref, br_ref, a2_ref, rp_ref,
                  xl_ref, xr_ref, xrf_ref):
    x = x_ref[...]
    xl = jnp.dot(x, wl_ref[...], preferred_element_type=jnp.float32) + bl_ref[...]
    xr = jnp.dot(x, wr_ref[...], preferred_element_type=jnp.float32) + br_ref[...]
    t = xl + xr
    lk = jnp.where(t >= 0, t, 0.2 * t)
    rp = rp_ref[...]
    xl_ref[...] = jnp.dot(xl, rp, preferred_element_type=jnp.float32)
    xr_ref[...] = (jnp.dot(xr, rp, preferred_element_type=jnp.float32)
                   + jnp.dot(lk, a2_ref[...],
                             preferred_element_type=jnp.float32))
    xrf_ref[...] = xr


def _dense_proj(x, Wl, bl, Wr, br, a2, rpad, xw, w, block=2000):
    n, d = x.shape
    hc = Wl.shape[1]
    grid = n // block
    body = _proj_l1_body if rpad is None else _proj_l2_body
    in_specs = [
        pl.BlockSpec((block, d), lambda i: (i, 0)),
        pl.BlockSpec((d, hc), lambda i: (0, 0)),
        pl.BlockSpec((1, hc), lambda i: (0, 0)),
        pl.BlockSpec((d, hc), lambda i: (0, 0)),
        pl.BlockSpec((1, hc), lambda i: (0, 0)),
        pl.BlockSpec((hc, 128), lambda i: (0, 0)),
    ]
    args = [x, Wl, bl.reshape(1, hc), Wr, br.reshape(1, hc), a2]
    if rpad is not None:
        in_specs.append(pl.BlockSpec((hc, xw), lambda i: (0, 0)))
        args.append(rpad)
    return pl.pallas_call(
        body,
        grid=(grid,),
        in_specs=in_specs,
        out_specs=[
            pl.BlockSpec((block, xw), lambda i: (i, 0)),
            pl.BlockSpec((block, w), lambda i: (i, 0)),
            pl.BlockSpec((block, hc), lambda i: (i, 0)),
        ],
        out_shape=[
            jax.ShapeDtypeStruct((n, xw), jnp.float32),
            jax.ShapeDtypeStruct((n, w), jnp.float32),
            jax.ShapeDtypeStruct((n, hc), jnp.float32),
        ],
    )(*args)


def _edge_l1_body(e2, b2, xl_ref, xr_ref, a2a_ref, a2b_ref, pa_ref, pb_ref,
                  p2_ref, out_ref):
    xlw = xl_ref[...]
    xrw = xr_ref[:, :256]
    ua = _unpack_lo(xlw) + _unpack_lo(xrw)
    ub = _unpack_hi(xlw) + _unpack_hi(xrw)
    s = _unpack_lo(xr_ref[:, 256:384])[:, :8]
    lka = jnp.where(ua >= 0, ua, 0.2 * ua)
    lkb = jnp.where(ub >= 0, ub, 0.2 * ub)
    alpha = (jnp.dot(lka, a2a_ref[...], preferred_element_type=jnp.float32)
             + jnp.dot(lkb, a2b_ref[...],
                       preferred_element_type=jnp.float32))[:, :8]
    row = pl.program_id(0) * b2 + lax.broadcasted_iota(jnp.int32, (b2, 1), 0)
    aex = jnp.where(row < e2, jnp.exp(alpha - s), 0.0)
    out_ref[:, :256] = ua * jnp.dot(aex, pa_ref[...],
                                    preferred_element_type=jnp.float32)
    out_ref[:, 256:512] = ub * jnp.dot(aex, pb_ref[...],
                                       preferred_element_type=jnp.float32)
    out_ref[:, 512:640] = jnp.dot(aex, p2_ref[...],
                                  preferred_element_type=jnp.float32)


def _edge_l2_body(e2, b2, xl_ref, xr_ref, a2_ref, p_ref, p2_ref, out_ref):
    u = xl_ref[...] + xr_ref[...]
    lk0 = u[:, :64]
    lk = jnp.where(lk0 >= 0, lk0, 0.2 * lk0)
    alpha = jnp.dot(lk, a2_ref[...], preferred_element_type=jnp.float32)[:, :1]
    s = xr_ref[:, 64:65]
    row = pl.program_id(0) * b2 + lax.broadcasted_iota(jnp.int32, (b2, 1), 0)
    aex = jnp.where(row < e2, jnp.exp(alpha - s), 0.0)
    out_ref[...] = (u * jnp.dot(aex, p_ref[...],
                                preferred_element_type=jnp.float32)
                    + jnp.dot(aex, p2_ref[...],
                              preferred_element_type=jnp.float32))


def _edge_math(xl_src, xr_g, mats, e2, heads, outw, b2=2048):
    e2p, xw = xl_src.shape
    w = xr_g.shape[1]
    grid = e2p // b2
    body = functools.partial(
        _edge_l1_body if heads == 8 else _edge_l2_body, e2, b2)
    return pl.pallas_call(
        body,
        grid=(grid,),
        in_specs=[
            pl.BlockSpec((b2, xw), lambda i: (i, 0)),
            pl.BlockSpec((b2, w), lambda i: (i, 0)),
        ] + [pl.BlockSpec(m.shape, lambda i: (0, 0)) for m in mats],
        out_specs=pl.BlockSpec((b2, outw), lambda i: (i, 0)),
        out_shape=jax.ShapeDtypeStruct((e2p, outw), jnp.float32),
    )(xl_src, xr_g, *mats)


def _sc_gather(table, idx2, pw):
    _, F = table.shape
    e2p = idx2.shape[0] * 128
    B = 128
    nb = pw // B
    irw = pw // 128

    @functools.partial(
        pl.kernel, mesh=_mesh(),
        out_type=jax.ShapeDtypeStruct((e2p, F), table.dtype),
        scratch_types=(
            [pltpu.VMEM((irw, 128), jnp.int32)]
            + [pltpu.VMEM((B, F), table.dtype) for _ in range(2)]
            + [pltpu.SemaphoreType.DMA for _ in range(4)]
        ),
    )
    def kfn(table_hbm, idx_hbm, out_hbm, idx_all, r0, r1, g0, g1, w0, w1):
        rows = [r0, r1]
        gsem = [g0, g1]
        wsem = [w0, w1]
        w = lax.axis_index("s") * _NC + lax.axis_index("c")
        base = w * pw
        pltpu.sync_copy(idx_hbm.at[pl.ds(w * irw, irw)], idx_all)

        def gstart(i, sl):
            pltpu.async_copy(table_hbm.at[idx_all.at[i]], rows[sl], gsem[sl])

        def gwait(sl):
            pltpu.make_async_copy(table_hbm.at[pl.ds(0, B)],
                                  rows[sl], gsem[sl]).wait()

        def wstart(i, sl):
            pltpu.async_copy(rows[sl], out_hbm.at[pl.ds(base + i * B, B)],
                             wsem[sl])

        def wwait(sl):
            pltpu.make_async_copy(table_hbm.at[pl.ds(0, B)],
                                  rows[sl], wsem[sl]).wait()

        gstart(0, 0)
        gstart(1, 1)

        @pl.loop(0, nb // 2 - 1)
        def _(t):
            i0 = 2 * t
            gwait(0)
            wstart(i0, 0)
            gwait(1)
            wstart(i0 + 1, 1)
            wwait(0)
            gstart(i0 + 2, 0)
            wwait(1)
            gstart(i0 + 3, 1)

        gwait(0)
        wstart(nb - 2, 0)
        gwait(1)
        wstart(nb - 1, 1)
        wwait(0)
        wwait(1)

    return kfn(table, idx2)


def _acc_copy(src, dst, s, nn):
    slab = (nn // _NS) & ~7
    pltpu.sync_copy(src.at[pl.ds(s * slab, slab)], dst.at[pl.ds(s * slab, slab)])
    tail = nn - _NS * slab
    if tail:
        @pl.when(s == 0)
        def _():
            pltpu.sync_copy(src.at[pl.ds(_NS * slab, tail)],
                            dst.at[pl.ds(_NS * slab, tail)])


def _sc_scatter(v, idx2, zc, nch, pw):
    nn = zc.shape[0]
    B = 128
    nb = pw // B
    irw = pw // 128

    @functools.partial(
        pl.kernel, mesh=_mesh(),
        out_type=jax.ShapeDtypeStruct((_NC * nch, nn, 128), jnp.float32),
        scratch_types=(
            [pltpu.VMEM((irw, 128), jnp.int32)]
            + [pltpu.VMEM((B, 128), jnp.float32) for _ in range(2)]
            + [pltpu.VMEM_SHARED((nn, 128), jnp.float32)]
            + [pltpu.SemaphoreType.DMA for _ in range(4)]
        ),
    )
    def kfn(v_hbm, idx_hbm, zc_hbm, agg_hbm, idx_all,
            r0, r1, acc_sh, l0, l1, s0, s1):
        rows = [r0, r1]
        lsem = [l0, l1]
        ssem = [s0, s1]
        c = lax.axis_index("c")
        s = lax.axis_index("s")
        w = s * _NC + c
        base = w * pw
        pltpu.sync_copy(idx_hbm.at[pl.ds(w * irw, irw)], idx_all)

        for ch in range(nch):
            _acc_copy(zc_hbm, acc_sh, s, nn)
            plsc.subcore_barrier()

            def lstart(i, sl):
                pltpu.async_copy(
                    v_hbm.at[pl.ds(base + i * B, B), pl.ds(ch * 128, 128)],
                    rows[sl], lsem[sl])

            def lwait(sl):
                pltpu.make_async_copy(
                    v_hbm.at[pl.ds(base, B), pl.ds(ch * 128, 128)],
                    rows[sl], lsem[sl]).wait()

            def sstart(i, sl):
                pltpu.async_copy(rows[sl], acc_sh.at[idx_all.at[i]],
                                 ssem[sl], add=True)

            def swait(sl):
                pltpu.make_async_copy(
                    v_hbm.at[pl.ds(base, B), pl.ds(ch * 128, 128)],
                    rows[sl], ssem[sl]).wait()

            lstart(0, 0)
            lstart(1, 1)

            @pl.loop(0, nb // 2 - 1)
            def _(t):
                i0 = 2 * t
                lwait(0)
                sstart(i0, 0)
                lwait(1)
                sstart(i0 + 1, 1)
                swait(0)
                lstart(i0 + 2, 0)
                swait(1)
                lstart(i0 + 3, 1)

            lwait(0)
            sstart(nb - 2, 0)
            lwait(1)
            sstart(nb - 1, 1)
            swait(0)
            swait(1)

            plsc.subcore_barrier()
            _acc_copy(acc_sh, agg_hbm.at[c * nch + ch], s, nn)

    return kfn(v, idx2, zc)


def _finish_body(nch, hc, heads, nblk, parts_ref, xr_ref, bias_ref,
                 pmat_ref, dsel_ref, t_ref, st_ref, acc):
    i = pl.program_id(0)
    den128 = parts_ref[nch - 1] + parts_ref[2 * nch - 1]
    den_h = jnp.dot(den128, dsel_ref[...], preferred_element_type=jnp.float32)
    denb = jnp.dot(den_h, pmat_ref[...], preferred_element_type=jnp.float32)
    ncols = hc // (nch - 1) if nch > 1 else hc

    @pl.when(i == 0)
    def _():
        acc[...] = jnp.zeros((2, hc), jnp.float32)

    for c in range(max(1, nch - 1)):
        sl = slice(c * ncols, (c + 1) * ncols)
        tt = ((parts_ref[c] + parts_ref[nch + c])[:, :ncols] / denb[:, sl]
              - xr_ref[:, sl] + bias_ref[:, sl])
        t_ref[:, sl] = tt
        acc[0:1, sl] += jnp.sum(tt, axis=0, keepdims=True)
        acc[1:2, sl] += jnp.sum(tt * tt, axis=0, keepdims=True)

    @pl.when(i == nblk - 1)
    def _():
        st_ref[...] = acc[...]


def _finish(parts, xr, bias, heads, ch, block=2000):
    nparts, n, _ = parts.shape
    nch = nparts // 2
    hc = heads * ch
    nblk = n // block
    pmat = jnp.repeat(jnp.eye(heads, dtype=jnp.float32), ch, axis=1)
    dcol = jnp.arange(heads) if heads == 8 else jnp.array([64])
    dsel = jnp.zeros((128, heads), jnp.float32).at[
        dcol, jnp.arange(heads)].set(1.0)
    return pl.pallas_call(
        functools.partial(_finish_body, nch, hc, heads, nblk),
        grid=(nblk,),
        in_specs=[
            pl.BlockSpec((nparts, block, 128), lambda i: (0, i, 0)),
            pl.BlockSpec((block, hc), lambda i: (i, 0)),
            pl.BlockSpec((1, hc), lambda i: (0, 0)),
            pl.BlockSpec((heads, hc), lambda i: (0, 0)),
            pl.BlockSpec((128, heads), lambda i: (0, 0)),
        ],
        out_specs=[
            pl.BlockSpec((block, hc), lambda i: (i, 0)),
            pl.BlockSpec((2, hc), lambda i: (0, 0)),
        ],
        out_shape=[
            jax.ShapeDtypeStruct((n, hc), jnp.float32),
            jax.ShapeDtypeStruct((2, hc), jnp.float32),
        ],
        scratch_shapes=[pltpu.VMEM((2, hc), jnp.float32)],
    )(parts, xr, bias.reshape(1, hc), pmat, dsel)


def _bn_gelu_body(n, t_ref, st_ref, g_ref, be_ref, o_ref):
    mu = st_ref[0:1] * (1.0 / n)
    var = st_ref[1:2] * (1.0 / n) - mu * mu
    y = (g_ref[...] * (t_ref[...] - mu) * jax.lax.rsqrt(var + 1e-5)
         + be_ref[...])
    o_ref[...] = 0.5 * y * (1.0 + jax.lax.erf(y * 0.7071067811865476))


def _bn_gelu(t, st, g, be, block=2000):
    n, hc = t.shape
    return pl.pallas_call(
        functools.partial(_bn_gelu_body, n),
        grid=(n // block,),
        in_specs=[
            pl.BlockSpec((block, hc), lambda i: (i, 0)),
            pl.BlockSpec((2, hc), lambda i: (0, 0)),
            pl.BlockSpec((1, hc), lambda i: (0, 0)),
            pl.BlockSpec((1, hc), lambda i: (0, 0)),
        ],
        out_specs=pl.BlockSpec((block, hc), lambda i: (i, 0)),
        out_shape=jax.ShapeDtypeStruct((n, hc), jnp.float32),
    )(t, st, g.reshape(1, hc), be.reshape(1, hc))


def _cls_body(h_ref, wc_ref, bc_ref, o_ref):
    z = jnp.dot(h_ref[...], wc_ref[...],
                preferred_element_type=jnp.float32) + bc_ref[...]
    m = jnp.max(z, axis=1, keepdims=True)
    o_ref[...] = z - m - jnp.log(jnp.sum(jnp.exp(z - m), axis=1,
                                         keepdims=True))


def _classifier(h, Wc, bc, block=2000):
    n, d = h.shape
    k = Wc.shape[1]
    return pl.pallas_call(
        _cls_body,
        grid=(n // block,),
        in_specs=[
            pl.BlockSpec((block, d), lambda i: (i, 0)),
            pl.BlockSpec((d, k), lambda i: (0, 0)),
            pl.BlockSpec((1, k), lambda i: (0, 0)),
        ],
        out_specs=pl.BlockSpec((block, k), lambda i: (i, 0)),
        out_shape=jax.ShapeDtypeStruct((n, k), jnp.float32),
    )(h, Wc, bc.reshape(1, k))


def _gat_layer(x, src2, dst2, e2, Wl, bl, Wr, br, att, bias, heads, ch):
    n = x.shape[0]
    hc = heads * ch
    e2p = src2.shape[0] * 128
    pw = e2p // _NW
    src = src2
    dst = dst2
    augw = -(hc + heads) % 128 + hc + heads

    af = att.reshape(hc)
    hrep = jnp.repeat(jnp.arange(heads), ch)
    if heads == 8:
        a2_proj = jnp.zeros((hc, 128), jnp.float32).at[
            jnp.arange(hc), hrep].set(af)
        mats = [a2_proj[:256], a2_proj[256:],
                jnp.repeat(jnp.eye(heads, dtype=jnp.float32), ch, axis=1)[:, :256],
                jnp.repeat(jnp.eye(heads, dtype=jnp.float32), ch, axis=1)[:, 256:],
                jnp.eye(heads, 128, dtype=jnp.float32)]
        rpad = None
        xw, ww = 256, 384
    else:
        a2_proj = jnp.zeros((hc, 128), jnp.float32).at[
            jnp.arange(hc), 64].set(af)
        a2_edge = jnp.zeros((hc, 128), jnp.float32).at[
            jnp.arange(hc), 0].set(af)
        mats = [a2_edge,
                (jnp.arange(128) < 64).astype(jnp.float32).reshape(1, 128),
                jnp.zeros((1, 128), jnp.float32).at[0, 64].set(1.0)]
        rpad = jnp.eye(hc, 128, dtype=jnp.float32)
        xw, ww = 128, 128

    xl_t, xr_aug, xr = _dense_proj(x, Wl, bl, Wr, br, a2_proj, rpad, xw, ww)

    xl_src = _sc_gather(xl_t, src, pw)
    xr_g = _sc_gather(xr_aug, dst, pw)

    v_aug = _edge_math(xl_src, xr_g, mats, e2, heads, augw)

    nch = augw // 128
    zc = jnp.zeros((n, 128), jnp.float32)
    parts = _sc_scatter(v_aug, dst2, zc, nch, pw)

    return _finish(parts, xr, bias, heads, ch)


def kernel(x, edge_index, Wl1, bl1, Wr1, br1, att1, bias1, g1, be1,
           Wl2, bl2, Wr2, br2, att2, bias2, g2, be2, Wc, bc):
    n = x.shape[0]
    e = edge_index.shape[1]
    e2 = e + n
    grain = _NW * _GRAIN
    e2p = ((e2 + grain - 1) // grain) * grain

    loop = jnp.arange(n, dtype=edge_index.dtype)
    pad = e2p - e2
    src2 = jnp.pad(jnp.concatenate([edge_index[0], loop]),
                   (0, pad)).reshape(e2p // 128, 128)
    dst2 = jnp.pad(jnp.concatenate([edge_index[1], loop]),
                   (0, pad)).reshape(e2p // 128, 128)

    t1, st1 = _gat_layer(x, src2, dst2, e2, Wl1, bl1, Wr1, br1,
                         att1, bias1, 8, 64)
    h = _bn_gelu(t1, st1, g1, be1)
    t2, st2 = _gat_layer(h, src2, dst2, e2, Wl2, bl2, Wr2, br2,
                         att2, bias2, 1, 64)
    h = _bn_gelu(t2, st2, g2, be2)
    return _classifier(h, Wc, bc)
